# initial kernel scaffold (unmeasured)
import jax
import jax.numpy as jnp
from jax import lax
from jax.experimental import pallas as pl
from jax.experimental.pallas import tpu as pltpu

N_DEV = 8
M_BLK = 512
N_TILE = 2048

_DeviceIdType = getattr(pl, "DeviceIdType", None) or pltpu.DeviceIdType
MESH = _DeviceIdType.MESH
_sem_signal = getattr(pl, "semaphore_signal", None) or pltpu.semaphore_signal
_sem_wait = getattr(pl, "semaphore_wait", None) or pltpu.semaphore_wait
_CompilerParams = getattr(pltpu, "CompilerParams", None) or pltpu.TPUCompilerParams


def kernel(x, w_mat):
    m, k = x.shape
    _, n = w_mat.shape
    assert m == N_DEV * M_BLK

    def body(x_ref, w_ref, out_ref, comm_ref, amax_ref,
             send_sems, recv_sems, ax_send_sems, ax_recv_sems, out_sem):
        d = lax.axis_index("i")
        left = lax.rem(d + N_DEV - 1, N_DEV)
        right = lax.rem(d + 1, N_DEV)

        barrier = pltpu.get_barrier_semaphore()
        for nbr in (left, right):
            _sem_signal(barrier, 1, device_id=(nbr,), device_id_type=MESH)
        _sem_wait(barrier, 2)

        def accum_block(slot, b, first):
            xb = x_ref[pl.ds(b * M_BLK, M_BLK), :]
            for t in range(n // N_TILE):
                sl = slice(t * N_TILE, (t + 1) * N_TILE)
                p = jnp.dot(xb, w_ref[:, sl], preferred_element_type=jnp.float32)
                if first:
                    comm_ref[slot, :, sl] = p
                else:
                    comm_ref[slot, :, sl] = comm_ref[slot, :, sl] + p

        for s in range(N_DEV - 1):
            slot = s % 2
            b = lax.rem(d + 2 * N_DEV - 1 - s, N_DEV)
            accum_block(slot, b, first=(s == 0))
            rdma = pltpu.make_async_remote_copy(
                src_ref=comm_ref.at[slot],
                dst_ref=comm_ref.at[1 - slot],
                send_sem=send_sems.at[slot],
                recv_sem=recv_sems.at[1 - slot],
                device_id=(right,),
                device_id_type=MESH,
            )
            rdma.start()
            rdma.wait()

        accum_block(1, d, first=False)

        local_amax = jnp.max(jnp.abs(comm_ref[1]))
        amax_ref[pl.ds(d, 1)] = jnp.full((1, 8, 128), local_amax, jnp.float32)
        sends = []
        for o in range(1, N_DEV):
            tgt = lax.rem(d + o, N_DEV)
            rdma = pltpu.make_async_remote_copy(
                src_ref=amax_ref.at[pl.ds(d, 1)],
                dst_ref=amax_ref.at[pl.ds(d, 1)],
                send_sem=ax_send_sems.at[o],
                recv_sem=ax_recv_sems.at[d],
                device_id=(tgt,),
                device_id_type=MESH,
            )
            rdma.start()
            sends.append(rdma)
        for o in range(1, N_DEV):
            src = lax.rem(d + N_DEV - o, N_DEV)
            rwait = pltpu.make_async_remote_copy(
                src_ref=amax_ref.at[pl.ds(d, 1)],
                dst_ref=amax_ref.at[pl.ds(d, 1)],
                send_sem=ax_send_sems.at[o],
                recv_sem=ax_recv_sems.at[src],
                device_id=(src,),
                device_id_type=MESH,
            )
            rwait.wait_recv()
        for rdma in sends:
            rdma.wait_send()

        g_amax = jnp.max(amax_ref[...])
        scale = g_amax / 127.0
        y = comm_ref[1]
        q = jnp.clip(jnp.round(y / scale), -127.0, 127.0)
        comm_ref[0] = q * scale

        cp = pltpu.make_async_copy(comm_ref.at[0], out_ref, out_sem)
        cp.start()
        cp.wait()

    return pl.pallas_call(
        body,
        out_shape=jax.ShapeDtypeStruct((M_BLK, n), jnp.float32),
        in_specs=[
            pl.BlockSpec(memory_space=pltpu.VMEM),
            pl.BlockSpec(memory_space=pltpu.VMEM),
        ],
        out_specs=pl.BlockSpec(memory_space=pltpu.ANY),
        scratch_shapes=[
            pltpu.VMEM((2, M_BLK, n), jnp.float32),
            pltpu.VMEM((N_DEV, 8, 128), jnp.float32),
            pltpu.SemaphoreType.DMA((2,)),
            pltpu.SemaphoreType.DMA((2,)),
            pltpu.SemaphoreType.DMA((N_DEV,)),
            pltpu.SemaphoreType.DMA((N_DEV,)),
            pltpu.SemaphoreType.DMA,
        ],
        compiler_params=_CompilerParams(
            collective_id=0,
            vmem_limit_bytes=64 * 1024 * 1024,
        ),
    )(x, w_mat)


# baseline (device time: 1357648 ns/iter reference)
import jax
import jax.numpy as jnp
from jax import lax
from jax.experimental import pallas as pl
from jax.experimental.pallas import tpu as pltpu

N_DEV = 8
M_BLK = 512
N_TILE = 2048

_DeviceIdType = getattr(pl, "DeviceIdType", None) or pltpu.DeviceIdType
MESH = _DeviceIdType.MESH
_sem_signal = getattr(pl, "semaphore_signal", None) or pltpu.semaphore_signal
_sem_wait = getattr(pl, "semaphore_wait", None) or pltpu.semaphore_wait
_CompilerParams = getattr(pltpu, "CompilerParams", None) or pltpu.TPUCompilerParams


def kernel(x, w_mat):
    m, k = x.shape
    _, n = w_mat.shape
    assert m == N_DEV * M_BLK

    def body(x_ref, w_ref, out_ref, comm_ref, xblk_ref, amax_ref,
             send_sems, recv_sems, ax_send_sems, ax_recv_sems, xblk_sem,
             out_sem):
        d = lax.axis_index("i")
        left = lax.rem(d + N_DEV - 1, N_DEV)
        right = lax.rem(d + 1, N_DEV)

        barrier = pltpu.get_barrier_semaphore()
        for nbr in (left, right):
            _sem_signal(barrier, 1, device_id=(nbr,), device_id_type=MESH)
        _sem_wait(barrier, 2)

        def accum_block(slot, b, first):
            cp = pltpu.make_async_copy(
                x_ref.at[pl.ds(b * M_BLK, M_BLK), :], xblk_ref, xblk_sem
            )
            cp.start()
            cp.wait()
            xb = xblk_ref[...]
            for t in range(n // N_TILE):
                sl = slice(t * N_TILE, (t + 1) * N_TILE)
                p = jnp.dot(xb, w_ref[:, sl], preferred_element_type=jnp.float32)
                if first:
                    comm_ref[slot, :, sl] = p
                else:
                    comm_ref[slot, :, sl] = comm_ref[slot, :, sl] + p

        for s in range(N_DEV - 1):
            slot = s % 2
            b = lax.rem(d + 2 * N_DEV - 1 - s, N_DEV)
            accum_block(slot, b, first=(s == 0))
            rdma = pltpu.make_async_remote_copy(
                src_ref=comm_ref.at[slot],
                dst_ref=comm_ref.at[1 - slot],
                send_sem=send_sems.at[slot],
                recv_sem=recv_sems.at[1 - slot],
                device_id=(right,),
                device_id_type=MESH,
            )
            rdma.start()
            rdma.wait()

        accum_block(1, d, first=False)

        local_amax = jnp.float32(0.0)
        for t in range(n // N_TILE):
            sl = slice(t * N_TILE, (t + 1) * N_TILE)
            local_amax = jnp.maximum(
                local_amax, jnp.max(jnp.abs(comm_ref[1, :, sl]))
            )
        amax_ref[pl.ds(d, 1)] = jnp.full((1, 8, 128), local_amax, jnp.float32)
        sends = []
        for o in range(1, N_DEV):
            tgt = lax.rem(d + o, N_DEV)
            rdma = pltpu.make_async_remote_copy(
                src_ref=amax_ref.at[pl.ds(d, 1)],
                dst_ref=amax_ref.at[pl.ds(d, 1)],
                send_sem=ax_send_sems.at[o],
                recv_sem=ax_recv_sems.at[d],
                device_id=(tgt,),
                device_id_type=MESH,
            )
            rdma.start()
            sends.append(rdma)
        for o in range(1, N_DEV):
            src = lax.rem(d + N_DEV - o, N_DEV)
            rwait = pltpu.make_async_remote_copy(
                src_ref=amax_ref.at[pl.ds(d, 1)],
                dst_ref=amax_ref.at[pl.ds(d, 1)],
                send_sem=ax_send_sems.at[o],
                recv_sem=ax_recv_sems.at[src],
                device_id=(src,),
                device_id_type=MESH,
            )
            rwait.wait_recv()
        for rdma in sends:
            rdma.wait_send()

        g_amax = jnp.max(amax_ref[...])
        scale = g_amax / 127.0
        inv_scale = 127.0 / g_amax
        for t in range(n // N_TILE):
            sl = slice(t * N_TILE, (t + 1) * N_TILE)
            q = jnp.clip(
                jnp.round(comm_ref[1, :, sl] * inv_scale), -127.0, 127.0
            )
            comm_ref[0, :, sl] = q * scale

        cp = pltpu.make_async_copy(comm_ref.at[0], out_ref, out_sem)
        cp.start()
        cp.wait()

    return pl.pallas_call(
        body,
        out_shape=jax.ShapeDtypeStruct((M_BLK, n), jnp.float32),
        in_specs=[
            pl.BlockSpec(memory_space=pl.ANY),
            pl.BlockSpec(memory_space=pltpu.VMEM),
        ],
        out_specs=pl.BlockSpec(memory_space=pl.ANY),
        scratch_shapes=[
            pltpu.VMEM((2, M_BLK, n), jnp.float32),
            pltpu.VMEM((M_BLK, k), jnp.float32),
            pltpu.VMEM((N_DEV, 8, 128), jnp.float32),
            pltpu.SemaphoreType.DMA((2,)),
            pltpu.SemaphoreType.DMA((2,)),
            pltpu.SemaphoreType.DMA((N_DEV,)),
            pltpu.SemaphoreType.DMA((N_DEV,)),
            pltpu.SemaphoreType.DMA,
            pltpu.SemaphoreType.DMA,
        ],
        compiler_params=_CompilerParams(
            collective_id=0,
            vmem_limit_bytes=64 * 1024 * 1024,
        ),
    )(x, w_mat)


# device time: 732059 ns/iter; 1.8546x vs baseline; 1.8546x over previous
import jax
import jax.numpy as jnp
from jax import lax
from jax.experimental import pallas as pl
from jax.experimental.pallas import tpu as pltpu

N_DEV = 8
M_BLK = 512
N_TILE = 2048

_DeviceIdType = getattr(pl, "DeviceIdType", None) or pltpu.DeviceIdType
MESH = _DeviceIdType.MESH
_sem_signal = getattr(pl, "semaphore_signal", None) or pltpu.semaphore_signal
_sem_wait = getattr(pl, "semaphore_wait", None) or pltpu.semaphore_wait
_CompilerParams = getattr(pltpu, "CompilerParams", None) or pltpu.TPUCompilerParams


def kernel(x, w_mat):
    m, k = x.shape
    _, n = w_mat.shape
    assert m == N_DEV * M_BLK
    nh = n // 2

    def body(x_ref, w_ref, out_ref, comm_ref, xblk_ref, amax_ref,
             send_sems, recv_sems, ax_send_sems, ax_recv_sems, xblk_sems,
             out_sem):
        d = lax.axis_index("i")
        left = lax.rem(d + N_DEV - 1, N_DEV)
        right = lax.rem(d + 1, N_DEV)

        barrier = pltpu.get_barrier_semaphore()
        for nbr in (left, right):
            _sem_signal(barrier, 1, device_id=(nbr,), device_id_type=MESH)
        _sem_wait(barrier, 2)

        def fetch_x(b, half):
            cp = pltpu.make_async_copy(
                x_ref.at[pl.ds(b * M_BLK, M_BLK), :],
                xblk_ref.at[half],
                xblk_sems.at[half],
            )
            cp.start()
            return cp

        def accum_half(slot, half, first):
            xb = xblk_ref[half]
            for t in range(nh // N_TILE):
                lo = half * nh + t * N_TILE
                sl = slice(lo, lo + N_TILE)
                p = jnp.dot(xb, w_ref[:, sl], preferred_element_type=jnp.float32)
                if first:
                    comm_ref[slot, :, sl] = p
                else:
                    comm_ref[slot, :, sl] = comm_ref[slot, :, sl] + p

        def ring_rdma(slot, half, tgt):
            lo = half * nh
            sl = slice(lo, lo + nh)
            return pltpu.make_async_remote_copy(
                src_ref=comm_ref.at[slot, :, sl],
                dst_ref=comm_ref.at[1 - slot, :, sl],
                send_sem=send_sems.at[slot, half],
                recv_sem=recv_sems.at[1 - slot, half],
                device_id=(tgt,),
                device_id_type=MESH,
            )

        for s in range(N_DEV - 1):
            slot = s % 2
            b_cw = lax.rem(d + 2 * N_DEV - 1 - s, N_DEV)
            b_ccw = lax.rem(d + 1 + s, N_DEV)
            cp0 = fetch_x(b_cw, 0)
            cp1 = fetch_x(b_ccw, 1)
            cp0.wait()
            accum_half(slot, 0, first=(s == 0))
            cp1.wait()
            accum_half(slot, 1, first=(s == 0))
            r_cw = ring_rdma(slot, 0, right)
            r_ccw = ring_rdma(slot, 1, left)
            r_cw.start()
            r_ccw.start()
            r_cw.wait()
            r_ccw.wait()

        cp0 = fetch_x(d, 0)
        cp0.wait()
        xblk_ref[1] = xblk_ref[0]
        accum_half(1, 0, first=False)
        accum_half(1, 1, first=False)

        local_amax = jnp.float32(0.0)
        for t in range(n // N_TILE):
            sl = slice(t * N_TILE, (t + 1) * N_TILE)
            local_amax = jnp.maximum(
                local_amax, jnp.max(jnp.abs(comm_ref[1, :, sl]))
            )
        amax_ref[pl.ds(d, 1)] = jnp.full((1, 8, 128), local_amax, jnp.float32)
        sends = []
        for o in range(1, N_DEV):
            tgt = lax.rem(d + o, N_DEV)
            rdma = pltpu.make_async_remote_copy(
                src_ref=amax_ref.at[pl.ds(d, 1)],
                dst_ref=amax_ref.at[pl.ds(d, 1)],
                send_sem=ax_send_sems.at[o],
                recv_sem=ax_recv_sems.at[d],
                device_id=(tgt,),
                device_id_type=MESH,
            )
            rdma.start()
            sends.append(rdma)
        for o in range(1, N_DEV):
            src = lax.rem(d + N_DEV - o, N_DEV)
            rwait = pltpu.make_async_remote_copy(
                src_ref=amax_ref.at[pl.ds(d, 1)],
                dst_ref=amax_ref.at[pl.ds(d, 1)],
                send_sem=ax_send_sems.at[o],
                recv_sem=ax_recv_sems.at[src],
                device_id=(src,),
                device_id_type=MESH,
            )
            rwait.wait_recv()
        for rdma in sends:
            rdma.wait_send()

        g_amax = jnp.max(amax_ref[...])
        scale = g_amax / 127.0
        inv_scale = 127.0 / g_amax
        for t in range(n // N_TILE):
            sl = slice(t * N_TILE, (t + 1) * N_TILE)
            q = jnp.clip(
                jnp.round(comm_ref[1, :, sl] * inv_scale), -127.0, 127.0
            )
            comm_ref[0, :, sl] = q * scale

        cp = pltpu.make_async_copy(comm_ref.at[0], out_ref, out_sem)
        cp.start()
        cp.wait()

    return pl.pallas_call(
        body,
        out_shape=jax.ShapeDtypeStruct((M_BLK, n), jnp.float32),
        in_specs=[
            pl.BlockSpec(memory_space=pl.ANY),
            pl.BlockSpec(memory_space=pltpu.VMEM),
        ],
        out_specs=pl.BlockSpec(memory_space=pl.ANY),
        scratch_shapes=[
            pltpu.VMEM((2, M_BLK, n), jnp.float32),
            pltpu.VMEM((2, M_BLK, k), jnp.float32),
            pltpu.VMEM((N_DEV, 8, 128), jnp.float32),
            pltpu.SemaphoreType.DMA((2, 2)),
            pltpu.SemaphoreType.DMA((2, 2)),
            pltpu.SemaphoreType.DMA((N_DEV,)),
            pltpu.SemaphoreType.DMA((N_DEV,)),
            pltpu.SemaphoreType.DMA((2,)),
            pltpu.SemaphoreType.DMA,
        ],
        compiler_params=_CompilerParams(
            collective_id=0,
            vmem_limit_bytes=64 * 1024 * 1024,
        ),
    )(x, w_mat)


# device time: 702550 ns/iter; 1.9325x vs baseline; 1.0420x over previous
import jax
import jax.numpy as jnp
from jax import lax
from jax.experimental import pallas as pl
from jax.experimental.pallas import tpu as pltpu

N_DEV = 8
M_BLK = 512
N_SUB = 2048
C_SUB = 2

_DeviceIdType = getattr(pl, "DeviceIdType", None) or pltpu.DeviceIdType
MESH = _DeviceIdType.MESH
_sem_signal = getattr(pl, "semaphore_signal", None) or pltpu.semaphore_signal
_sem_wait = getattr(pl, "semaphore_wait", None) or pltpu.semaphore_wait
_CompilerParams = getattr(pltpu, "CompilerParams", None) or pltpu.TPUCompilerParams


def kernel(x, w_mat):
    m, k = x.shape
    _, n = w_mat.shape
    assert m == N_DEV * M_BLK
    nh = n // 2
    assert nh == C_SUB * N_SUB

    def body(x_ref, w_ref, out_ref, comm_ref, xblk_ref, amax_ref,
             send_sems, recv_sems, ax_send_sems, ax_recv_sems, xblk_sems,
             out_sems):
        d = lax.axis_index("i")
        left = lax.rem(d + N_DEV - 1, N_DEV)
        right = lax.rem(d + 1, N_DEV)

        barrier = pltpu.get_barrier_semaphore()
        for nbr in (left, right):
            _sem_signal(barrier, 1, device_id=(nbr,), device_id_type=MESH)
        _sem_wait(barrier, 2)

        def fetch_x(b, parity, half):
            cp = pltpu.make_async_copy(
                x_ref.at[pl.ds(b * M_BLK, M_BLK), :],
                xblk_ref.at[parity, half],
                xblk_sems.at[parity, half],
            )
            cp.start()
            return cp

        def sub_slice(half, c):
            lo = half * nh + c * N_SUB
            return slice(lo, lo + N_SUB)

        def ring_rdma(slot, half, c, tgt):
            sl = sub_slice(half, c)
            return pltpu.make_async_remote_copy(
                src_ref=comm_ref.at[slot, :, sl],
                dst_ref=comm_ref.at[1 - slot, :, sl],
                send_sem=send_sems.at[slot, half, c],
                recv_sem=recv_sems.at[1 - slot, half, c],
                device_id=(tgt,),
                device_id_type=MESH,
            )

        def blocks_for(s):
            b_cw = lax.rem(d + 2 * N_DEV - 1 - s, N_DEV)
            b_ccw = lax.rem(d + 1 + s, N_DEV)
            return b_cw, b_ccw

        b_cw, b_ccw = blocks_for(0)
        xcps = [fetch_x(b_cw, 0, 0), fetch_x(b_ccw, 0, 1)]

        for s in range(N_DEV - 1):
            slot = s % 2
            par = s % 2
            for h in (0, 1):
                xcps[h].wait()
            sends = []
            for c in range(C_SUB):
                for h in (0, 1):
                    sl = sub_slice(h, c)
                    p = jnp.dot(
                        xblk_ref[par, h], w_ref[:, sl],
                        preferred_element_type=jnp.float32,
                    )
                    if s == 0:
                        comm_ref[slot, :, sl] = p
                    else:
                        comm_ref[slot, :, sl] = comm_ref[slot, :, sl] + p
                    rdma = ring_rdma(slot, h, c, right if h == 0 else left)
                    rdma.start()
                    sends.append(rdma)
            if s < N_DEV - 2:
                b_cw, b_ccw = blocks_for(s + 1)
                xcps = [fetch_x(b_cw, (s + 1) % 2, 0),
                        fetch_x(b_ccw, (s + 1) % 2, 1)]
            else:
                xcps = [fetch_x(d, (s + 1) % 2, 0)]
            for rdma in sends:
                rdma.wait()

        xcps[0].wait()
        xb = xblk_ref[1, 0]
        local_amax = jnp.float32(0.0)
        for c in range(C_SUB):
            for h in (0, 1):
                sl = sub_slice(h, c)
                p = jnp.dot(xb, w_ref[:, sl],
                            preferred_element_type=jnp.float32)
                res = comm_ref[1, :, sl] + p
                local_amax = jnp.maximum(local_amax, jnp.max(jnp.abs(res)))
                comm_ref[1, :, sl] = res

        amax_ref[pl.ds(d, 1)] = jnp.full((1, 8, 128), local_amax, jnp.float32)
        ax_sends = []
        for o in range(1, N_DEV):
            tgt = lax.rem(d + o, N_DEV)
            rdma = pltpu.make_async_remote_copy(
                src_ref=amax_ref.at[pl.ds(d, 1)],
                dst_ref=amax_ref.at[pl.ds(d, 1)],
                send_sem=ax_send_sems.at[o],
                recv_sem=ax_recv_sems.at[d],
                device_id=(tgt,),
                device_id_type=MESH,
            )
            rdma.start()
            ax_sends.append(rdma)
        for o in range(1, N_DEV):
            src = lax.rem(d + N_DEV - o, N_DEV)
            rwait = pltpu.make_async_remote_copy(
                src_ref=amax_ref.at[pl.ds(d, 1)],
                dst_ref=amax_ref.at[pl.ds(d, 1)],
                send_sem=ax_send_sems.at[o],
                recv_sem=ax_recv_sems.at[src],
                device_id=(src,),
                device_id_type=MESH,
            )
            rwait.wait_recv()
        for rdma in ax_sends:
            rdma.wait_send()

        g_amax = jnp.max(amax_ref[...])
        scale = g_amax / 127.0
        inv_scale = 127.0 / g_amax
        out_cps = []
        for t in range(n // N_SUB):
            sl = slice(t * N_SUB, (t + 1) * N_SUB)
            q = jnp.clip(
                jnp.round(comm_ref[1, :, sl] * inv_scale), -127.0, 127.0
            )
            comm_ref[0, :, sl] = q * scale
            cp = pltpu.make_async_copy(
                comm_ref.at[0, :, sl], out_ref.at[:, sl], out_sems.at[t]
            )
            cp.start()
            out_cps.append(cp)
        for cp in out_cps:
            cp.wait()

    return pl.pallas_call(
        body,
        out_shape=jax.ShapeDtypeStruct((M_BLK, n), jnp.float32),
        in_specs=[
            pl.BlockSpec(memory_space=pl.ANY),
            pl.BlockSpec(memory_space=pltpu.VMEM),
        ],
        out_specs=pl.BlockSpec(memory_space=pl.ANY),
        scratch_shapes=[
            pltpu.VMEM((2, M_BLK, n), jnp.float32),
            pltpu.VMEM((2, 2, M_BLK, k), jnp.float32),
            pltpu.VMEM((N_DEV, 8, 128), jnp.float32),
            pltpu.SemaphoreType.DMA((2, 2, C_SUB)),
            pltpu.SemaphoreType.DMA((2, 2, C_SUB)),
            pltpu.SemaphoreType.DMA((N_DEV,)),
            pltpu.SemaphoreType.DMA((N_DEV,)),
            pltpu.SemaphoreType.DMA((2, 2)),
            pltpu.SemaphoreType.DMA((4,)),
        ],
        compiler_params=_CompilerParams(
            collective_id=0,
            vmem_limit_bytes=64 * 1024 * 1024,
        ),
    )(x, w_mat)


# device time: 688100 ns/iter; 1.9730x vs baseline; 1.0210x over previous
import jax
import jax.numpy as jnp
from jax import lax
from jax.experimental import pallas as pl
from jax.experimental.pallas import tpu as pltpu

N_DEV = 8
M_BLK = 512
N_SUB = 2048
C_SUB = 2

_DeviceIdType = getattr(pl, "DeviceIdType", None) or pltpu.DeviceIdType
MESH = _DeviceIdType.MESH
_sem_signal = getattr(pl, "semaphore_signal", None) or pltpu.semaphore_signal
_sem_wait = getattr(pl, "semaphore_wait", None) or pltpu.semaphore_wait
_CompilerParams = getattr(pltpu, "CompilerParams", None) or pltpu.TPUCompilerParams


def kernel(x, w_mat):
    m, k = x.shape
    _, n = w_mat.shape
    assert m == N_DEV * M_BLK
    nh = n // 2
    assert nh == C_SUB * N_SUB

    def body(x_ref, w_ref, out_ref, comm_ref, xblk_ref, pstage_ref, amax_ref,
             send_sems, recv_sems, ax_send_sems, ax_recv_sems, xblk_sems,
             out_sems):
        d = lax.axis_index("i")
        left = lax.rem(d + N_DEV - 1, N_DEV)
        right = lax.rem(d + 1, N_DEV)

        barrier = pltpu.get_barrier_semaphore()
        for nbr in (left, right):
            _sem_signal(barrier, 1, device_id=(nbr,), device_id_type=MESH)
        _sem_wait(barrier, 2)

        def fetch_x(b, parity, half):
            cp = pltpu.make_async_copy(
                x_ref.at[pl.ds(b * M_BLK, M_BLK), :],
                xblk_ref.at[parity, half],
                xblk_sems.at[parity, half],
            )
            cp.start()
            return cp

        def sub_slice(half, c):
            lo = half * nh + c * N_SUB
            return slice(lo, lo + N_SUB)

        def ring_rdma(slot, half, c, tgt):
            sl = sub_slice(half, c)
            return pltpu.make_async_remote_copy(
                src_ref=comm_ref.at[slot, :, sl],
                dst_ref=comm_ref.at[1 - slot, :, sl],
                send_sem=send_sems.at[slot, half, c],
                recv_sem=recv_sems.at[1 - slot, half, c],
                device_id=(tgt,),
                device_id_type=MESH,
            )

        def blocks_for(s):
            b_cw = lax.rem(d + 2 * N_DEV - 1 - s, N_DEV)
            b_ccw = lax.rem(d + 1 + s, N_DEV)
            return b_cw, b_ccw

        b_cw, b_ccw = blocks_for(0)
        xcps = [fetch_x(b_cw, 0, 0), fetch_x(b_ccw, 0, 1)]
        for h in (0, 1):
            xcps[h].wait()
        for h in (0, 1):
            pstage_ref[h] = jnp.dot(
                xblk_ref[0, h], w_ref[:, sub_slice(h, 0)],
                preferred_element_type=jnp.float32,
            )

        for s in range(N_DEV - 1):
            slot = s % 2
            par = s % 2
            sends = []
            for c in range(C_SUB):
                for h in (0, 1):
                    sl = sub_slice(h, c)
                    if c == 0:
                        p = pstage_ref[h]
                    else:
                        p = jnp.dot(
                            xblk_ref[par, h], w_ref[:, sl],
                            preferred_element_type=jnp.float32,
                        )
                    if s == 0:
                        comm_ref[slot, :, sl] = p
                    else:
                        comm_ref[slot, :, sl] = comm_ref[slot, :, sl] + p
                    rdma = ring_rdma(slot, h, c, right if h == 0 else left)
                    rdma.start()
                    sends.append(rdma)
            npar = (s + 1) % 2
            if s < N_DEV - 2:
                b_cw, b_ccw = blocks_for(s + 1)
                xcps = [fetch_x(b_cw, npar, 0), fetch_x(b_ccw, npar, 1)]
                xcps[0].wait()
                xcps[1].wait()
                halves = (0, 1)
            else:
                cp = fetch_x(d, npar, 0)
                cp.wait()
                xblk_ref[npar, 1] = xblk_ref[npar, 0]
                halves = (0, 1)
            for h in halves:
                pstage_ref[h] = jnp.dot(
                    xblk_ref[npar, h], w_ref[:, sub_slice(h, 0)],
                    preferred_element_type=jnp.float32,
                )
            for rdma in sends:
                rdma.wait()

        local_amax = jnp.float32(0.0)
        for c in range(C_SUB):
            for h in (0, 1):
                sl = sub_slice(h, c)
                if c == 0:
                    p = pstage_ref[h]
                else:
                    p = jnp.dot(xblk_ref[1, h], w_ref[:, sl],
                                preferred_element_type=jnp.float32)
                res = comm_ref[1, :, sl] + p
                local_amax = jnp.maximum(local_amax, jnp.max(jnp.abs(res)))
                comm_ref[1, :, sl] = res

        amax_ref[pl.ds(d, 1)] = jnp.full((1, 8, 128), local_amax, jnp.float32)
        ax_sends = []
        for o in range(1, N_DEV):
            tgt = lax.rem(d + o, N_DEV)
            rdma = pltpu.make_async_remote_copy(
                src_ref=amax_ref.at[pl.ds(d, 1)],
                dst_ref=amax_ref.at[pl.ds(d, 1)],
                send_sem=ax_send_sems.at[o],
                recv_sem=ax_recv_sems.at[d],
                device_id=(tgt,),
                device_id_type=MESH,
            )
            rdma.start()
            ax_sends.append(rdma)
        for o in range(1, N_DEV):
            src = lax.rem(d + N_DEV - o, N_DEV)
            rwait = pltpu.make_async_remote_copy(
                src_ref=amax_ref.at[pl.ds(d, 1)],
                dst_ref=amax_ref.at[pl.ds(d, 1)],
                send_sem=ax_send_sems.at[o],
                recv_sem=ax_recv_sems.at[src],
                device_id=(src,),
                device_id_type=MESH,
            )
            rwait.wait_recv()
        for rdma in ax_sends:
            rdma.wait_send()

        g_amax = jnp.max(amax_ref[...])
        scale = g_amax / 127.0
        inv_scale = 127.0 / g_amax
        out_cps = []
        for t in range(n // N_SUB):
            sl = slice(t * N_SUB, (t + 1) * N_SUB)
            q = jnp.clip(
                jnp.round(comm_ref[1, :, sl] * inv_scale), -127.0, 127.0
            )
            comm_ref[0, :, sl] = q * scale
            cp = pltpu.make_async_copy(
                comm_ref.at[0, :, sl], out_ref.at[:, sl], out_sems.at[t]
            )
            cp.start()
            out_cps.append(cp)
        for cp in out_cps:
            cp.wait()

    return pl.pallas_call(
        body,
        out_shape=jax.ShapeDtypeStruct((M_BLK, n), jnp.float32),
        in_specs=[
            pl.BlockSpec(memory_space=pl.ANY),
            pl.BlockSpec(memory_space=pltpu.VMEM),
        ],
        out_specs=pl.BlockSpec(memory_space=pl.ANY),
        scratch_shapes=[
            pltpu.VMEM((2, M_BLK, n), jnp.float32),
            pltpu.VMEM((2, 2, M_BLK, k), jnp.float32),
            pltpu.VMEM((2, M_BLK, N_SUB), jnp.float32),
            pltpu.VMEM((N_DEV, 8, 128), jnp.float32),
            pltpu.SemaphoreType.DMA((2, 2, C_SUB)),
            pltpu.SemaphoreType.DMA((2, 2, C_SUB)),
            pltpu.SemaphoreType.DMA((N_DEV,)),
            pltpu.SemaphoreType.DMA((N_DEV,)),
            pltpu.SemaphoreType.DMA((2, 2)),
            pltpu.SemaphoreType.DMA((4,)),
        ],
        compiler_params=_CompilerParams(
            collective_id=0,
            vmem_limit_bytes=64 * 1024 * 1024,
        ),
    )(x, w_mat)


# device time: 670557 ns/iter; 2.0247x vs baseline; 1.0262x over previous
import jax
import jax.numpy as jnp
from jax import lax
from jax.experimental import pallas as pl
from jax.experimental.pallas import tpu as pltpu

N_DEV = 8
M_BLK = 512
N_SUB = 2048
C_SUB = 2

_DeviceIdType = getattr(pl, "DeviceIdType", None) or pltpu.DeviceIdType
MESH = _DeviceIdType.MESH
_sem_signal = getattr(pl, "semaphore_signal", None) or pltpu.semaphore_signal
_sem_wait = getattr(pl, "semaphore_wait", None) or pltpu.semaphore_wait
_CompilerParams = getattr(pltpu, "CompilerParams", None) or pltpu.TPUCompilerParams


def kernel(x, w_mat):
    m, k = x.shape
    _, n = w_mat.shape
    assert m == N_DEV * M_BLK
    nh = n // 2
    assert nh == C_SUB * N_SUB

    def body(x_ref, w_ref, out_ref, comm_ref, xblk_ref, pstage_ref, amax_ref,
             send_sems, recv_sems, credit_sems, ax_send_sems, ax_recv_sems,
             xblk_sems, out_sems):
        d = lax.axis_index("i")
        left = lax.rem(d + N_DEV - 1, N_DEV)
        right = lax.rem(d + 1, N_DEV)

        barrier = pltpu.get_barrier_semaphore()
        for nbr in (left, right):
            _sem_signal(barrier, 1, device_id=(nbr,), device_id_type=MESH)
        _sem_wait(barrier, 2)

        def fetch_x(b, parity, half):
            cp = pltpu.make_async_copy(
                x_ref.at[pl.ds(b * M_BLK, M_BLK), :],
                xblk_ref.at[parity, half],
                xblk_sems.at[parity, half],
            )
            cp.start()
            return cp

        def sub_slice(half, c):
            lo = half * nh + c * N_SUB
            return slice(lo, lo + N_SUB)

        def ring_rdma(slot, half, c, tgt):
            sl = sub_slice(half, c)
            return pltpu.make_async_remote_copy(
                src_ref=comm_ref.at[slot, :, sl],
                dst_ref=comm_ref.at[1 - slot, :, sl],
                send_sem=send_sems.at[slot, half, c],
                recv_sem=recv_sems.at[1 - slot, half, c],
                device_id=(tgt,),
                device_id_type=MESH,
            )

        def blocks_for(s):
            b_cw = lax.rem(d + 2 * N_DEV - 1 - s, N_DEV)
            b_ccw = lax.rem(d + 1 + s, N_DEV)
            return b_cw, b_ccw

        b_cw, b_ccw = blocks_for(0)
        xcps = [fetch_x(b_cw, 0, 0), fetch_x(b_ccw, 0, 1)]
        for h in (0, 1):
            xcps[h].wait()
        for h in (0, 1):
            pstage_ref[h] = jnp.dot(
                xblk_ref[0, h], w_ref[:, sub_slice(h, 0)],
                preferred_element_type=jnp.float32,
            )

        sends = {}
        for s in range(N_DEV - 1):
            slot = s % 2
            par = s % 2
            for c in range(C_SUB):
                for h in (0, 1):
                    sl = sub_slice(h, c)
                    upstream = left if h == 0 else right
                    if s > 0:
                        ring_rdma(1 - slot, h, c, right if h == 0 else left
                                  ).wait_recv()
                    if s >= 2:
                        sends[(s - 2, h, c)].wait_send()
                    if c == 0:
                        p = pstage_ref[h]
                    else:
                        p = jnp.dot(
                            xblk_ref[par, h], w_ref[:, sl],
                            preferred_element_type=jnp.float32,
                        )
                    if s == 0:
                        comm_ref[slot, :, sl] = p
                    else:
                        comm_ref[slot, :, sl] = comm_ref[slot, :, sl] + p
                    if 1 <= s <= 5:
                        _sem_signal(credit_sems.at[slot, h, c], 1,
                                    device_id=(upstream,),
                                    device_id_type=MESH)
                    if s >= 2:
                        _sem_wait(credit_sems.at[(s - 1) % 2, h, c], 1)
                    rdma = ring_rdma(slot, h, c, right if h == 0 else left)
                    rdma.start()
                    sends[(s, h, c)] = rdma
            npar = (s + 1) % 2
            if s < N_DEV - 2:
                b_cw, b_ccw = blocks_for(s + 1)
                xcps = [fetch_x(b_cw, npar, 0), fetch_x(b_ccw, npar, 1)]
                xcps[0].wait()
                xcps[1].wait()
            else:
                cp = fetch_x(d, npar, 0)
                cp.wait()
                xblk_ref[npar, 1] = xblk_ref[npar, 0]
            for h in (0, 1):
                pstage_ref[h] = jnp.dot(
                    xblk_ref[npar, h], w_ref[:, sub_slice(h, 0)],
                    preferred_element_type=jnp.float32,
                )

        local_amax = jnp.float32(0.0)
        for c in range(C_SUB):
            for h in (0, 1):
                sl = sub_slice(h, c)
                ring_rdma(0, h, c, right if h == 0 else left).wait_recv()
                sends[(5, h, c)].wait_send()
                if c == 0:
                    p = pstage_ref[h]
                else:
                    p = jnp.dot(xblk_ref[1, h], w_ref[:, sl],
                                preferred_element_type=jnp.float32)
                res = comm_ref[1, :, sl] + p
                local_amax = jnp.maximum(local_amax, jnp.max(jnp.abs(res)))
                comm_ref[1, :, sl] = res

        amax_ref[pl.ds(d, 1)] = jnp.full((1, 8, 128), local_amax, jnp.float32)
        ax_sends = []
        for o in range(1, N_DEV):
            tgt = lax.rem(d + o, N_DEV)
            rdma = pltpu.make_async_remote_copy(
                src_ref=amax_ref.at[pl.ds(d, 1)],
                dst_ref=amax_ref.at[pl.ds(d, 1)],
                send_sem=ax_send_sems.at[o],
                recv_sem=ax_recv_sems.at[d],
                device_id=(tgt,),
                device_id_type=MESH,
            )
            rdma.start()
            ax_sends.append(rdma)
        for c in range(C_SUB):
            for h in (0, 1):
                sends[(6, h, c)].wait_send()
        for o in range(1, N_DEV):
            src = lax.rem(d + N_DEV - o, N_DEV)
            rwait = pltpu.make_async_remote_copy(
                src_ref=amax_ref.at[pl.ds(d, 1)],
                dst_ref=amax_ref.at[pl.ds(d, 1)],
                send_sem=ax_send_sems.at[o],
                recv_sem=ax_recv_sems.at[src],
                device_id=(src,),
                device_id_type=MESH,
            )
            rwait.wait_recv()
        for rdma in ax_sends:
            rdma.wait_send()

        g_amax = jnp.max(amax_ref[...])
        scale = g_amax / 127.0
        inv_scale = 127.0 / g_amax
        out_cps = []
        for t in range(n // N_SUB):
            sl = slice(t * N_SUB, (t + 1) * N_SUB)
            q = jnp.clip(
                jnp.round(comm_ref[1, :, sl] * inv_scale), -127.0, 127.0
            )
            comm_ref[0, :, sl] = q * scale
            cp = pltpu.make_async_copy(
                comm_ref.at[0, :, sl], out_ref.at[:, sl], out_sems.at[t]
            )
            cp.start()
            out_cps.append(cp)
        for cp in out_cps:
            cp.wait()

    return pl.pallas_call(
        body,
        out_shape=jax.ShapeDtypeStruct((M_BLK, n), jnp.float32),
        in_specs=[
            pl.BlockSpec(memory_space=pl.ANY),
            pl.BlockSpec(memory_space=pltpu.VMEM),
        ],
        out_specs=pl.BlockSpec(memory_space=pl.ANY),
        scratch_shapes=[
            pltpu.VMEM((2, M_BLK, n), jnp.float32),
            pltpu.VMEM((2, 2, M_BLK, k), jnp.float32),
            pltpu.VMEM((2, M_BLK, N_SUB), jnp.float32),
            pltpu.VMEM((N_DEV, 8, 128), jnp.float32),
            pltpu.SemaphoreType.DMA((2, 2, C_SUB)),
            pltpu.SemaphoreType.DMA((2, 2, C_SUB)),
            pltpu.SemaphoreType.REGULAR((2, 2, C_SUB)),
            pltpu.SemaphoreType.DMA((N_DEV,)),
            pltpu.SemaphoreType.DMA((N_DEV,)),
            pltpu.SemaphoreType.DMA((2, 2)),
            pltpu.SemaphoreType.DMA((4,)),
        ],
        compiler_params=_CompilerParams(
            collective_id=0,
            vmem_limit_bytes=64 * 1024 * 1024,
        ),
    )(x, w_mat)


# device time: 668415 ns/iter; 2.0311x vs baseline; 1.0032x over previous
import jax
import jax.numpy as jnp
from jax import lax
from jax.experimental import pallas as pl
from jax.experimental.pallas import tpu as pltpu

N_DEV = 8
M_BLK = 512
N_SUB = 1024
C_SUB = 4
N_QT = 2048

_DeviceIdType = getattr(pl, "DeviceIdType", None) or pltpu.DeviceIdType
MESH = _DeviceIdType.MESH
_sem_signal = getattr(pl, "semaphore_signal", None) or pltpu.semaphore_signal
_sem_wait = getattr(pl, "semaphore_wait", None) or pltpu.semaphore_wait
_CompilerParams = getattr(pltpu, "CompilerParams", None) or pltpu.TPUCompilerParams


def kernel(x, w_mat):
    m, k = x.shape
    _, n = w_mat.shape
    assert m == N_DEV * M_BLK
    nh = n // 2
    assert nh == C_SUB * N_SUB

    def body(x_ref, w_ref, out_ref, comm_ref, xblk_ref, pstage_ref, amax_ref,
             send_sems, recv_sems, credit_sems, ax_send_sems, ax_recv_sems,
             xblk_sems, out_sems):
        d = lax.axis_index("i")
        left = lax.rem(d + N_DEV - 1, N_DEV)
        right = lax.rem(d + 1, N_DEV)

        barrier = pltpu.get_barrier_semaphore()
        for nbr in (left, right):
            _sem_signal(barrier, 1, device_id=(nbr,), device_id_type=MESH)
        _sem_wait(barrier, 2)

        def fetch_x(b, parity, half):
            cp = pltpu.make_async_copy(
                x_ref.at[pl.ds(b * M_BLK, M_BLK), :],
                xblk_ref.at[parity, half],
                xblk_sems.at[parity, half],
            )
            cp.start()
            return cp

        def sub_slice(half, c):
            lo = half * nh + c * N_SUB
            return slice(lo, lo + N_SUB)

        def ring_rdma(slot, half, c, tgt):
            sl = sub_slice(half, c)
            return pltpu.make_async_remote_copy(
                src_ref=comm_ref.at[slot, :, sl],
                dst_ref=comm_ref.at[1 - slot, :, sl],
                send_sem=send_sems.at[slot, half, c],
                recv_sem=recv_sems.at[1 - slot, half, c],
                device_id=(tgt,),
                device_id_type=MESH,
            )

        def blocks_for(s):
            b_cw = lax.rem(d + 2 * N_DEV - 1 - s, N_DEV)
            b_ccw = lax.rem(d + 1 + s, N_DEV)
            return b_cw, b_ccw

        b_cw, b_ccw = blocks_for(0)
        xcps = [fetch_x(b_cw, 0, 0), fetch_x(b_ccw, 0, 1)]
        for h in (0, 1):
            xcps[h].wait()
        for h in (0, 1):
            pstage_ref[h] = jnp.dot(
                xblk_ref[0, h], w_ref[:, sub_slice(h, 0)],
                preferred_element_type=jnp.float32,
            )

        sends = {}
        for s in range(N_DEV - 1):
            slot = s % 2
            par = s % 2
            for c in range(C_SUB):
                for h in (0, 1):
                    sl = sub_slice(h, c)
                    upstream = left if h == 0 else right
                    if s > 0:
                        ring_rdma(1 - slot, h, c, right if h == 0 else left
                                  ).wait_recv()
                    if s >= 2:
                        sends[(s - 2, h, c)].wait_send()
                    if c == 0:
                        p = pstage_ref[h]
                    else:
                        p = jnp.dot(
                            xblk_ref[par, h], w_ref[:, sl],
                            preferred_element_type=jnp.float32,
                        )
                    if s == 0:
                        comm_ref[slot, :, sl] = p
                    else:
                        comm_ref[slot, :, sl] = comm_ref[slot, :, sl] + p
                    if 1 <= s <= 5:
                        _sem_signal(credit_sems.at[slot, h, c], 1,
                                    device_id=(upstream,),
                                    device_id_type=MESH)
                    if s >= 2:
                        _sem_wait(credit_sems.at[(s - 1) % 2, h, c], 1)
                    rdma = ring_rdma(slot, h, c, right if h == 0 else left)
                    rdma.start()
                    sends[(s, h, c)] = rdma
            npar = (s + 1) % 2
            if s < N_DEV - 2:
                b_cw, b_ccw = blocks_for(s + 1)
                xcps = [fetch_x(b_cw, npar, 0), fetch_x(b_ccw, npar, 1)]
                xcps[0].wait()
                xcps[1].wait()
            else:
                cp = fetch_x(d, npar, 0)
                cp.wait()
                xblk_ref[npar, 1] = xblk_ref[npar, 0]
            for h in (0, 1):
                pstage_ref[h] = jnp.dot(
                    xblk_ref[npar, h], w_ref[:, sub_slice(h, 0)],
                    preferred_element_type=jnp.float32,
                )

        local_amax = jnp.float32(0.0)
        for c in range(C_SUB):
            for h in (0, 1):
                sl = sub_slice(h, c)
                ring_rdma(0, h, c, right if h == 0 else left).wait_recv()
                sends[(5, h, c)].wait_send()
                if c == 0:
                    p = pstage_ref[h]
                else:
                    p = jnp.dot(xblk_ref[1, h], w_ref[:, sl],
                                preferred_element_type=jnp.float32)
                res = comm_ref[1, :, sl] + p
                local_amax = jnp.maximum(local_amax, jnp.max(jnp.abs(res)))
                comm_ref[1, :, sl] = res

        amax_ref[pl.ds(d, 1)] = jnp.full((1, 8, 128), local_amax, jnp.float32)
        ax_sends = []
        for o in range(1, N_DEV):
            tgt = lax.rem(d + o, N_DEV)
            rdma = pltpu.make_async_remote_copy(
                src_ref=amax_ref.at[pl.ds(d, 1)],
                dst_ref=amax_ref.at[pl.ds(d, 1)],
                send_sem=ax_send_sems.at[o],
                recv_sem=ax_recv_sems.at[d],
                device_id=(tgt,),
                device_id_type=MESH,
            )
            rdma.start()
            ax_sends.append(rdma)
        for c in range(C_SUB):
            for h in (0, 1):
                sends[(6, h, c)].wait_send()
        for o in range(1, N_DEV):
            src = lax.rem(d + N_DEV - o, N_DEV)
            rwait = pltpu.make_async_remote_copy(
                src_ref=amax_ref.at[pl.ds(d, 1)],
                dst_ref=amax_ref.at[pl.ds(d, 1)],
                send_sem=ax_send_sems.at[o],
                recv_sem=ax_recv_sems.at[src],
                device_id=(src,),
                device_id_type=MESH,
            )
            rwait.wait_recv()
        for rdma in ax_sends:
            rdma.wait_send()

        g_amax = jnp.max(amax_ref[...])
        scale = g_amax / 127.0
        inv_scale = 127.0 / g_amax
        out_cps = []
        for t in range(n // N_QT):
            sl = slice(t * N_QT, (t + 1) * N_QT)
            q = jnp.clip(
                jnp.round(comm_ref[1, :, sl] * inv_scale), -127.0, 127.0
            )
            comm_ref[0, :, sl] = q * scale
            cp = pltpu.make_async_copy(
                comm_ref.at[0, :, sl], out_ref.at[:, sl], out_sems.at[t]
            )
            cp.start()
            out_cps.append(cp)
        for cp in out_cps:
            cp.wait()

    return pl.pallas_call(
        body,
        out_shape=jax.ShapeDtypeStruct((M_BLK, n), jnp.float32),
        in_specs=[
            pl.BlockSpec(memory_space=pl.ANY),
            pl.BlockSpec(memory_space=pltpu.VMEM),
        ],
        out_specs=pl.BlockSpec(memory_space=pl.ANY),
        scratch_shapes=[
            pltpu.VMEM((2, M_BLK, n), jnp.float32),
            pltpu.VMEM((2, 2, M_BLK, k), jnp.float32),
            pltpu.VMEM((2, M_BLK, N_SUB), jnp.float32),
            pltpu.VMEM((N_DEV, 8, 128), jnp.float32),
            pltpu.SemaphoreType.DMA((2, 2, C_SUB)),
            pltpu.SemaphoreType.DMA((2, 2, C_SUB)),
            pltpu.SemaphoreType.REGULAR((2, 2, C_SUB)),
            pltpu.SemaphoreType.DMA((N_DEV,)),
            pltpu.SemaphoreType.DMA((N_DEV,)),
            pltpu.SemaphoreType.DMA((2, 2)),
            pltpu.SemaphoreType.DMA((4,)),
        ],
        compiler_params=_CompilerParams(
            collective_id=0,
            vmem_limit_bytes=64 * 1024 * 1024,
        ),
    )(x, w_mat)


# device time: 353675 ns/iter; 3.8387x vs baseline; 1.8899x over previous
import jax
import jax.numpy as jnp
from jax import lax
from jax.experimental import pallas as pl
from jax.experimental.pallas import tpu as pltpu

N_DEV = 8
M_BLK = 512
N_SUB = 1024
C_SUB = 4
N_QT = 2048

_DeviceIdType = getattr(pl, "DeviceIdType", None) or pltpu.DeviceIdType
MESH = _DeviceIdType.MESH
_sem_signal = getattr(pl, "semaphore_signal", None) or pltpu.semaphore_signal
_sem_wait = getattr(pl, "semaphore_wait", None) or pltpu.semaphore_wait
_CompilerParams = getattr(pltpu, "CompilerParams", None) or pltpu.TPUCompilerParams


def kernel(x, w_mat):
    m, k = x.shape
    _, n = w_mat.shape
    assert m == N_DEV * M_BLK
    nh = n // 2
    assert nh == C_SUB * N_SUB

    def body(x_ref, w_ref, out_ref, comm_ref, fres_ref, xblk_ref, pstage_ref,
             amax_ref, send_sems, recv_sems, credit_sems, ax_send_sems,
             ax_recv_sems, xblk_sems, out_sems):
        d = lax.axis_index("i")
        left = lax.rem(d + N_DEV - 1, N_DEV)
        right = lax.rem(d + 1, N_DEV)

        barrier = pltpu.get_barrier_semaphore()
        for nbr in (left, right):
            _sem_signal(barrier, 1, device_id=(nbr,), device_id_type=MESH)
        _sem_wait(barrier, 2)

        def fetch_x(b, parity, half):
            cp = pltpu.make_async_copy(
                x_ref.at[pl.ds(b * M_BLK, M_BLK), :],
                xblk_ref.at[parity, half],
                xblk_sems.at[parity, half],
            )
            cp.start()
            return cp

        def sub_slice(half, c):
            lo = half * nh + c * N_SUB
            return slice(lo, lo + N_SUB)

        def ring_rdma(slot, half, c, tgt):
            sl = sub_slice(half, c)
            return pltpu.make_async_remote_copy(
                src_ref=comm_ref.at[slot, :, sl],
                dst_ref=comm_ref.at[1 - slot, :, sl],
                send_sem=send_sems.at[slot, half, c],
                recv_sem=recv_sems.at[1 - slot, half, c],
                device_id=(tgt,),
                device_id_type=MESH,
            )

        def blocks_for(s):
            b_cw = lax.rem(d + 2 * N_DEV - 1 - s, N_DEV)
            b_ccw = lax.rem(d + 1 + s, N_DEV)
            return b_cw, b_ccw

        b_cw, b_ccw = blocks_for(0)
        xcps = [fetch_x(b_cw, 0, 0), fetch_x(b_ccw, 0, 1)]
        for h in (0, 1):
            xcps[h].wait()
        for h in (0, 1):
            pstage_ref[h] = jnp.dot(
                xblk_ref[0, h], w_ref[:, sub_slice(h, 0)],
                preferred_element_type=jnp.float32,
            )

        sends = {}
        for s in range(N_DEV - 1):
            slot = s % 2
            par = s % 2
            for c in range(C_SUB):
                for h in (0, 1):
                    sl = sub_slice(h, c)
                    upstream = left if h == 0 else right
                    if s > 0:
                        ring_rdma(1 - slot, h, c, right if h == 0 else left
                                  ).wait_recv()
                    if s >= 2:
                        sends[(s - 2, h, c)].wait_send()
                    if c == 0:
                        p = pstage_ref[h]
                    else:
                        p = jnp.dot(
                            xblk_ref[par, h], w_ref[:, sl],
                            preferred_element_type=jnp.float32,
                        )
                    if s == 0:
                        comm_ref[slot, :, sl] = p.astype(jnp.bfloat16)
                    else:
                        comm_ref[slot, :, sl] = (
                            comm_ref[slot, :, sl].astype(jnp.float32) + p
                        ).astype(jnp.bfloat16)
                    if 1 <= s <= 5:
                        _sem_signal(credit_sems.at[slot, h, c], 1,
                                    device_id=(upstream,),
                                    device_id_type=MESH)
                    if s >= 2:
                        _sem_wait(credit_sems.at[(s - 1) % 2, h, c], 1)
                    rdma = ring_rdma(slot, h, c, right if h == 0 else left)
                    rdma.start()
                    sends[(s, h, c)] = rdma
            npar = (s + 1) % 2
            if s < N_DEV - 2:
                b_cw, b_ccw = blocks_for(s + 1)
                xcps = [fetch_x(b_cw, npar, 0), fetch_x(b_ccw, npar, 1)]
                xcps[0].wait()
                xcps[1].wait()
            else:
                cp = fetch_x(d, npar, 0)
                cp.wait()
                xblk_ref[npar, 1] = xblk_ref[npar, 0]
            for h in (0, 1):
                pstage_ref[h] = jnp.dot(
                    xblk_ref[npar, h], w_ref[:, sub_slice(h, 0)],
                    preferred_element_type=jnp.float32,
                )

        local_amax = jnp.float32(0.0)
        for c in range(C_SUB):
            for h in (0, 1):
                sl = sub_slice(h, c)
                ring_rdma(0, h, c, right if h == 0 else left).wait_recv()
                sends[(5, h, c)].wait_send()
                if c == 0:
                    p = pstage_ref[h]
                else:
                    p = jnp.dot(xblk_ref[1, h], w_ref[:, sl],
                                preferred_element_type=jnp.float32)
                res = comm_ref[1, :, sl].astype(jnp.float32) + p
                local_amax = jnp.maximum(local_amax, jnp.max(jnp.abs(res)))
                fres_ref[:, sl] = res

        amax_ref[pl.ds(d, 1)] = jnp.full((1, 8, 128), local_amax, jnp.float32)
        ax_sends = []
        for o in range(1, N_DEV):
            tgt = lax.rem(d + o, N_DEV)
            rdma = pltpu.make_async_remote_copy(
                src_ref=amax_ref.at[pl.ds(d, 1)],
                dst_ref=amax_ref.at[pl.ds(d, 1)],
                send_sem=ax_send_sems.at[o],
                recv_sem=ax_recv_sems.at[d],
                device_id=(tgt,),
                device_id_type=MESH,
            )
            rdma.start()
            ax_sends.append(rdma)
        for c in range(C_SUB):
            for h in (0, 1):
                sends[(6, h, c)].wait_send()
        for o in range(1, N_DEV):
            src = lax.rem(d + N_DEV - o, N_DEV)
            rwait = pltpu.make_async_remote_copy(
                src_ref=amax_ref.at[pl.ds(d, 1)],
                dst_ref=amax_ref.at[pl.ds(d, 1)],
                send_sem=ax_send_sems.at[o],
                recv_sem=ax_recv_sems.at[src],
                device_id=(src,),
                device_id_type=MESH,
            )
            rwait.wait_recv()
        for rdma in ax_sends:
            rdma.wait_send()

        g_amax = jnp.max(amax_ref[...])
        scale = g_amax / 127.0
        inv_scale = 127.0 / g_amax
        out_cps = []
        for t in range(n // N_QT):
            sl = slice(t * N_QT, (t + 1) * N_QT)
            q = jnp.clip(
                jnp.round(fres_ref[:, sl] * inv_scale), -127.0, 127.0
            )
            fres_ref[:, sl] = q * scale
            cp = pltpu.make_async_copy(
                fres_ref.at[:, sl], out_ref.at[:, sl], out_sems.at[t]
            )
            cp.start()
            out_cps.append(cp)
        for cp in out_cps:
            cp.wait()

    return pl.pallas_call(
        body,
        out_shape=jax.ShapeDtypeStruct((M_BLK, n), jnp.float32),
        in_specs=[
            pl.BlockSpec(memory_space=pl.ANY),
            pl.BlockSpec(memory_space=pltpu.VMEM),
        ],
        out_specs=pl.BlockSpec(memory_space=pl.ANY),
        scratch_shapes=[
            pltpu.VMEM((2, M_BLK, n), jnp.bfloat16),
            pltpu.VMEM((M_BLK, n), jnp.float32),
            pltpu.VMEM((2, 2, M_BLK, k), jnp.float32),
            pltpu.VMEM((2, M_BLK, N_SUB), jnp.float32),
            pltpu.VMEM((N_DEV, 8, 128), jnp.float32),
            pltpu.SemaphoreType.DMA((2, 2, C_SUB)),
            pltpu.SemaphoreType.DMA((2, 2, C_SUB)),
            pltpu.SemaphoreType.REGULAR((2, 2, C_SUB)),
            pltpu.SemaphoreType.DMA((N_DEV,)),
            pltpu.SemaphoreType.DMA((N_DEV,)),
            pltpu.SemaphoreType.DMA((2, 2)),
            pltpu.SemaphoreType.DMA((4,)),
        ],
        compiler_params=_CompilerParams(
            collective_id=0,
            vmem_limit_bytes=64 * 1024 * 1024,
        ),
    )(x, w_mat)
